# Initial kernel scaffold; baseline (speedup 1.0000x reference)
import jax
import jax.numpy as jnp
from jax.experimental import pallas as pl


def kernel(x, edge_index, edge_attr, batch, enc_W, enc_b, edge_W1, edge_b1, edge_W2, edge_b2, root_W, conv_b, rf1_W1, rf1_b1, rf1_W2, rf1_b2, rf2_W1, rf2_b1, rf2_W2, rf2_b2):
    raise NotImplementedError("write your pallas kernel here")


# speedup vs baseline: 4.0204x; 4.0204x over previous
import functools

import jax
import jax.numpy as jnp
import numpy as np
from jax import lax
from jax.experimental import pallas as pl
from jax.experimental.pallas import tpu as pltpu
from jax.experimental.pallas import tpu_sc as plsc

N = 10000
E = 160000
DF = 128
DE = 16
H = 16
T = 16
NG = 64
ML = 64
RL = 64
NUP = 2

NC = 2
NS = 16
NW = NC * NS
EPW = E // NW
RPW = N // NS

TE = 4000
TN = 2000

_mesh = plsc.VectorSubcoreMesh(
    core_axis_name="c", subcore_axis_name="s", num_cores=NC, num_subcores=NS
)


@functools.partial(
    pl.kernel,
    out_type=jax.ShapeDtypeStruct((E, H), jnp.float32),
    mesh=_mesh,
    compiler_params=pltpu.CompilerParams(use_tc_tiling_on_sc=False),
    scratch_types=[
        pltpu.VMEM((EPW,), jnp.int32),
        pltpu.VMEM((EPW, H), jnp.float32),
        pltpu.SemaphoreType.DMA,
    ],
)
def _sc_gather(h_hbm, src_hbm, out_hbm, idx_v, rows_v, sem):
    wid = lax.axis_index("s") * NC + lax.axis_index("c")
    base = wid * EPW
    pltpu.sync_copy(src_hbm.at[pl.ds(base, EPW)], idx_v)
    pltpu.async_copy(h_hbm.at[idx_v], rows_v, sem).wait()
    pltpu.sync_copy(rows_v, out_hbm.at[pl.ds(base, EPW)])


@functools.partial(
    pl.kernel,
    out_type=jax.ShapeDtypeStruct((NC, N, H), jnp.float32),
    mesh=_mesh,
    compiler_params=pltpu.CompilerParams(use_tc_tiling_on_sc=False),
    scratch_types=[
        pltpu.VMEM((EPW,), jnp.int32),
        pltpu.VMEM((EPW, H), jnp.float32),
        pltpu.VMEM_SHARED((N, H), jnp.float32),
        pltpu.SemaphoreType.DMA,
    ],
)
def _sc_scatter(msg_hbm, dst_hbm, zeros_hbm, out_hbm, idx_v, rows_v, acc_sh, sem):
    c = lax.axis_index("c")
    s = lax.axis_index("s")
    base = (s * NC + c) * EPW
    pltpu.sync_copy(zeros_hbm.at[pl.ds(s * RPW, RPW)], acc_sh.at[pl.ds(s * RPW, RPW)])
    plsc.subcore_barrier()
    pltpu.sync_copy(dst_hbm.at[pl.ds(base, EPW)], idx_v)
    pltpu.sync_copy(msg_hbm.at[pl.ds(base, EPW)], rows_v)
    pltpu.sync_copy(rows_v, acc_sh.at[idx_v], add=True)
    plsc.subcore_barrier()
    pltpu.sync_copy(acc_sh.at[pl.ds(s * RPW, RPW)], out_hbm.at[c, pl.ds(s * RPW, RPW)])


def _enc_body(x_ref, w_ref, b_ref, o_ref):
    o_ref[...] = x_ref[...] @ w_ref[...] + b_ref[...]


def _enc(x, enc_W, enc_b):
    return pl.pallas_call(
        _enc_body,
        grid=(N // TN,),
        in_specs=[
            pl.BlockSpec((TN, DF), lambda i: (i, 0)),
            pl.BlockSpec((DF, H), lambda i: (0, 0)),
            pl.BlockSpec((1, H), lambda i: (0, 0)),
        ],
        out_specs=pl.BlockSpec((TN, H), lambda i: (i, 0)),
        out_shape=jax.ShapeDtypeStruct((N, H), jnp.float32),
    )(x, enc_W, enc_b.reshape(1, H))


def _msg_body(attr_ref, hg_ref, w1_ref, b1_ref, w2_ref, b2_ref, q_ref, r_ref, o_ref):
    z = jnp.maximum(attr_ref[...] @ w1_ref[...] + b1_ref[...], 0.0)
    w = z @ w2_ref[...] + b2_ref[...]
    hrep = hg_ref[...] @ q_ref[...]
    o_ref[...] = (hrep * w) @ r_ref[...]


def _msg(attr, hg, w1, b1, w2, b2, q, r):
    return pl.pallas_call(
        _msg_body,
        grid=(E // TE,),
        in_specs=[
            pl.BlockSpec((TE, DE), lambda i: (i, 0)),
            pl.BlockSpec((TE, H), lambda i: (i, 0)),
            pl.BlockSpec((DE, ML), lambda i: (0, 0)),
            pl.BlockSpec((1, ML), lambda i: (0, 0)),
            pl.BlockSpec((ML, H * H), lambda i: (0, 0)),
            pl.BlockSpec((1, H * H), lambda i: (0, 0)),
            pl.BlockSpec((H, H * H), lambda i: (0, 0)),
            pl.BlockSpec((H * H, H), lambda i: (0, 0)),
        ],
        out_specs=pl.BlockSpec((TE, H), lambda i: (i, 0)),
        out_shape=jax.ShapeDtypeStruct((E, H), jnp.float32),
    )(attr, hg, w1, b1, w2, b2, q, r)


def _upd_body(p0_ref, p1_ref, h_ref, w_ref, b_ref, o_ref):
    o_ref[...] = p0_ref[...] + p1_ref[...] + h_ref[...] @ w_ref[...] + b_ref[...]


def _upd(p0, p1, h, root_W, conv_b):
    return pl.pallas_call(
        _upd_body,
        grid=(N // TN,),
        in_specs=[
            pl.BlockSpec((TN, H), lambda i: (i, 0)),
            pl.BlockSpec((TN, H), lambda i: (i, 0)),
            pl.BlockSpec((TN, H), lambda i: (i, 0)),
            pl.BlockSpec((H, H), lambda i: (0, 0)),
            pl.BlockSpec((1, H), lambda i: (0, 0)),
        ],
        out_specs=pl.BlockSpec((TN, H), lambda i: (i, 0)),
        out_shape=jax.ShapeDtypeStruct((N, H), jnp.float32),
    )(p0, p1, h, root_W, conv_b.reshape(1, H))


def _readout_body(h0_ref, h1_ref, p0_ref, p1_ref, bf_ref,
                  rw_ref, rb_ref,
                  w1a_ref, w1b_ref, b1_ref, w1c_ref, b1c_ref,
                  w2a_ref, b2_ref, w2c_ref, b2c_ref,
                  o_ref):
    i = pl.program_id(0)
    h0 = h0_ref[...]
    h2 = p0_ref[...] + p1_ref[...] + h1_ref[...] @ rw_ref[...] + rb_ref[...]
    t1 = jnp.maximum(h0 @ w1a_ref[...] + h2 @ w1b_ref[...] + b1_ref[...], 0.0)
    r1 = t1 @ w1c_ref[...] + b1c_ref[...]
    t2 = jnp.maximum(h2 @ w2a_ref[...] + b2_ref[...], 0.0)
    r2 = t2 @ w2c_ref[...] + b2c_ref[...]
    res = (1.0 / (1.0 + jnp.exp(-r1))) * r2
    mask = (jnp.sum(h0, axis=1, keepdims=True) > 0.0).astype(jnp.float32)
    res = mask * res
    gids = lax.broadcasted_iota(jnp.int32, (TN, NG), 1).astype(jnp.float32)
    onehot = (bf_ref[...] == gids).astype(jnp.float32)
    part = lax.dot_general(onehot, res, (((0,), (0,)), ((), ())),
                           preferred_element_type=jnp.float32)

    @pl.when(i == 0)
    def _():
        o_ref[...] = part

    @pl.when(i != 0)
    def _():
        o_ref[...] = o_ref[...] + part


def _readout(h0, h1, p0, p1, batchf, root_W, conv_b,
             rf1_W1, rf1_b1, rf1_W2, rf1_b2, rf2_W1, rf2_b1, rf2_W2, rf2_b2):
    row = lambda i: (i, 0)
    fix = lambda i: (0, 0)
    return pl.pallas_call(
        _readout_body,
        grid=(N // TN,),
        in_specs=[
            pl.BlockSpec((TN, H), row),
            pl.BlockSpec((TN, H), row),
            pl.BlockSpec((TN, H), row),
            pl.BlockSpec((TN, H), row),
            pl.BlockSpec((TN, 1), row),
            pl.BlockSpec((H, H), fix),
            pl.BlockSpec((1, H), fix),
            pl.BlockSpec((H, RL), fix),
            pl.BlockSpec((H, RL), fix),
            pl.BlockSpec((1, RL), fix),
            pl.BlockSpec((RL, T), fix),
            pl.BlockSpec((1, T), fix),
            pl.BlockSpec((H, RL), fix),
            pl.BlockSpec((1, RL), fix),
            pl.BlockSpec((RL, T), fix),
            pl.BlockSpec((1, T), fix),
        ],
        out_specs=pl.BlockSpec((NG, T), fix),
        out_shape=jax.ShapeDtypeStruct((NG, T), jnp.float32),
    )(h0, h1, p0, p1, batchf,
      root_W, conv_b.reshape(1, H),
      rf1_W1[:H], rf1_W1[H:], rf1_b1.reshape(1, RL), rf1_W2, rf1_b2.reshape(1, T),
      rf2_W1, rf2_b1.reshape(1, RL), rf2_W2, rf2_b2.reshape(1, T))


@jax.jit
def kernel(x, edge_index, edge_attr, batch,
           enc_W, enc_b,
           edge_W1, edge_b1, edge_W2, edge_b2,
           root_W, conv_b,
           rf1_W1, rf1_b1, rf1_W2, rf1_b2,
           rf2_W1, rf2_b1, rf2_W2, rf2_b2):
    src = edge_index[0]
    dst = edge_index[1]
    batchf = batch.astype(jnp.float32).reshape(N, 1)
    zeros_nh = jnp.zeros((N, H), jnp.float32)
    q_sel = jnp.kron(jnp.eye(H, dtype=jnp.float32), jnp.ones((1, H), jnp.float32))
    r_sel = jnp.kron(jnp.ones((H, 1), jnp.float32), jnp.eye(H, dtype=jnp.float32))
    b1r = edge_b1.reshape(1, ML)
    b2r = edge_b2.reshape(1, H * H)

    h0 = _enc(x, enc_W, enc_b)

    hg = _sc_gather(h0, src)
    msg = _msg(edge_attr, hg, edge_W1, b1r, edge_W2, b2r, q_sel, r_sel)
    part = _sc_scatter(msg, dst, zeros_nh)
    h1 = _upd(part[0], part[1], h0, root_W, conv_b)

    hg2 = _sc_gather(h1, src)
    msg2 = _msg(edge_attr, hg2, edge_W1, b1r, edge_W2, b2r, q_sel, r_sel)
    part2 = _sc_scatter(msg2, dst, zeros_nh)

    return _readout(h0, h1, part2[0], part2[1], batchf, root_W, conv_b,
                    rf1_W1, rf1_b1, rf1_W2, rf1_b2,
                    rf2_W1, rf2_b1, rf2_W2, rf2_b2)


# speedup vs baseline: 6.2191x; 1.5469x over previous
import functools

import jax
import jax.numpy as jnp
import numpy as np
from jax import lax
from jax.experimental import pallas as pl
from jax.experimental.pallas import tpu as pltpu
from jax.experimental.pallas import tpu_sc as plsc

N = 10000
E = 160000
DF = 128
DE = 16
H = 16
T = 16
NG = 64
ML = 64
RL = 64

NC = 2
NS = 16
NW = NC * NS
EPW = E // NW
RPW = N // NS

PK = 128 // H
TE = 3200
TB = TE // PK
NB = N // PK
EB = E // PK

_mesh = plsc.VectorSubcoreMesh(
    core_axis_name="c", subcore_axis_name="s", num_cores=NC, num_subcores=NS
)


def _unpack(xf, k):
    return jnp.concatenate([xf[:, 16 * b:16 * (b + 1)] for b in range(PK)], axis=0)


def _pack(x, k):
    return jnp.concatenate([x[b * k:(b + 1) * k, :] for b in range(PK)], axis=1)


@functools.partial(
    pl.kernel,
    out_type=jax.ShapeDtypeStruct((E, H), jnp.float32),
    mesh=_mesh,
    compiler_params=pltpu.CompilerParams(use_tc_tiling_on_sc=False),
    scratch_types=[
        pltpu.VMEM((EPW,), jnp.int32),
        pltpu.VMEM((EPW, H), jnp.float32),
        pltpu.SemaphoreType.DMA,
    ],
)
def _sc_gather(h_hbm, src_hbm, out_hbm, idx_v, rows_v, sem):
    wid = lax.axis_index("s") * NC + lax.axis_index("c")
    base = wid * EPW
    pltpu.sync_copy(src_hbm.at[pl.ds(base, EPW)], idx_v)
    pltpu.async_copy(h_hbm.at[idx_v], rows_v, sem).wait()
    pltpu.sync_copy(rows_v, out_hbm.at[pl.ds(base, EPW)])


@functools.partial(
    pl.kernel,
    out_type=jax.ShapeDtypeStruct((NC, N, H), jnp.float32),
    mesh=_mesh,
    compiler_params=pltpu.CompilerParams(use_tc_tiling_on_sc=False),
    scratch_types=[
        pltpu.VMEM((EPW,), jnp.int32),
        pltpu.VMEM((EPW, H), jnp.float32),
        pltpu.VMEM_SHARED((N, H), jnp.float32),
        pltpu.SemaphoreType.DMA,
    ],
)
def _sc_scatter(msg_hbm, dst_hbm, zeros_hbm, out_hbm, idx_v, rows_v, acc_sh, sem):
    c = lax.axis_index("c")
    s = lax.axis_index("s")
    base = (s * NC + c) * EPW
    pltpu.sync_copy(zeros_hbm.at[pl.ds(s * RPW, RPW)], acc_sh.at[pl.ds(s * RPW, RPW)])
    plsc.subcore_barrier()
    pltpu.sync_copy(dst_hbm.at[pl.ds(base, EPW)], idx_v)
    pltpu.sync_copy(msg_hbm.at[pl.ds(base, EPW)], rows_v)
    pltpu.sync_copy(rows_v, acc_sh.at[idx_v], add=True)
    plsc.subcore_barrier()
    pltpu.sync_copy(acc_sh.at[pl.ds(s * RPW, RPW)], out_hbm.at[c, pl.ds(s * RPW, RPW)])


def _enc_body(xf_ref, wbd_ref, bf_ref, o_ref):
    o_ref[...] = xf_ref[...] @ wbd_ref[...] + bf_ref[...]


def _enc(xf, enc_Wbd, enc_bf):
    return pl.pallas_call(
        _enc_body,
        grid=(1,),
        in_specs=[
            pl.BlockSpec((NB, PK * DF), lambda i: (0, 0)),
            pl.BlockSpec((PK * DF, 128), lambda i: (0, 0)),
            pl.BlockSpec((1, 128), lambda i: (0, 0)),
        ],
        out_specs=pl.BlockSpec((NB, 128), lambda i: (0, 0)),
        out_shape=jax.ShapeDtypeStruct((NB, 128), jnp.float32),
    )(xf, enc_Wbd, enc_bf)


def _msg_body(af_ref, hgf_ref, w1_ref, b1_ref, w2_ref, b2_ref, q_ref, r_ref, o_ref):
    attr = _unpack(af_ref[...], TB)
    hg = _unpack(hgf_ref[...], TB)
    z = jnp.maximum(attr @ w1_ref[...] + b1_ref[...], 0.0)
    w = z @ w2_ref[...] + b2_ref[...]
    hrep = hg @ q_ref[...]
    msg = (hrep * w) @ r_ref[...]
    o_ref[...] = _pack(msg, TB)


def _msg(attrf, hgf, w1, b1, w2, b2, q, r):
    return pl.pallas_call(
        _msg_body,
        grid=(E // TE,),
        in_specs=[
            pl.BlockSpec((TB, 128), lambda i: (i, 0)),
            pl.BlockSpec((TB, 128), lambda i: (i, 0)),
            pl.BlockSpec((DE, ML), lambda i: (0, 0)),
            pl.BlockSpec((1, ML), lambda i: (0, 0)),
            pl.BlockSpec((ML, H * H), lambda i: (0, 0)),
            pl.BlockSpec((1, H * H), lambda i: (0, 0)),
            pl.BlockSpec((H, H * H), lambda i: (0, 0)),
            pl.BlockSpec((H * H, H), lambda i: (0, 0)),
        ],
        out_specs=pl.BlockSpec((TB, 128), lambda i: (i, 0)),
        out_shape=jax.ShapeDtypeStruct((EB, 128), jnp.float32),
    )(attrf, hgf, w1, b1, w2, b2, q, r)


def _upd_body(p0_ref, p1_ref, hf_ref, wbd_ref, bf_ref, o_ref):
    o_ref[...] = (p0_ref[...] + p1_ref[...]
                  + hf_ref[...] @ wbd_ref[...] + bf_ref[...])


def _upd(p0f, p1f, hf, root_Wbd, conv_bf):
    fix = lambda i: (0, 0)
    return pl.pallas_call(
        _upd_body,
        grid=(1,),
        in_specs=[
            pl.BlockSpec((NB, 128), fix),
            pl.BlockSpec((NB, 128), fix),
            pl.BlockSpec((NB, 128), fix),
            pl.BlockSpec((128, 128), fix),
            pl.BlockSpec((1, 128), fix),
        ],
        out_specs=pl.BlockSpec((NB, 128), fix),
        out_shape=jax.ShapeDtypeStruct((NB, 128), jnp.float32),
    )(p0f, p1f, hf, root_Wbd, conv_bf)


def _readout_body(h0_ref, h1_ref, p0_ref, p1_ref, bw_ref,
                  rw_ref, rb_ref,
                  w1a_ref, w1b_ref, b1_ref, w1c_ref, b1c_ref,
                  w2a_ref, b2_ref, w2c_ref, b2c_ref,
                  o_ref):
    h0f = h0_ref[...]
    h2f = p0_ref[...] + p1_ref[...] + h1_ref[...] @ rw_ref[...] + rb_ref[...]
    t1 = jnp.maximum(h0f @ w1a_ref[...] + h2f @ w1b_ref[...] + b1_ref[...], 0.0)
    r1 = t1 @ w1c_ref[...] + b1c_ref[...]
    t2 = jnp.maximum(h2f @ w2a_ref[...] + b2_ref[...], 0.0)
    r2 = t2 @ w2c_ref[...] + b2c_ref[...]
    resf = (1.0 / (1.0 + jnp.exp(-r1))) * r2
    resw = _unpack(resf, NB)
    h0w = _unpack(h0_ref[...], NB)
    maskw = (jnp.sum(h0w, axis=1, keepdims=True) > 0.0).astype(jnp.float32)
    resw = resw * maskw
    gids = lax.broadcasted_iota(jnp.int32, (N, NG), 1).astype(jnp.float32)
    onehot = (bw_ref[...] == gids).astype(jnp.float32)
    o_ref[...] = lax.dot_general(onehot, resw, (((0,), (0,)), ((), ())),
                                 preferred_element_type=jnp.float32)


def _readout(h0f, h1f, p0f, p1f, batchw,
             root_Wbd, conv_bf,
             w1abd, w1bbd, b1f, w1cbd, b1cf,
             w2abd, b2f, w2cbd, b2cf):
    fix = lambda i: (0, 0)
    return pl.pallas_call(
        _readout_body,
        grid=(1,),
        in_specs=[
            pl.BlockSpec((NB, 128), fix),
            pl.BlockSpec((NB, 128), fix),
            pl.BlockSpec((NB, 128), fix),
            pl.BlockSpec((NB, 128), fix),
            pl.BlockSpec((N, 1), fix),
            pl.BlockSpec((128, 128), fix),
            pl.BlockSpec((1, 128), fix),
            pl.BlockSpec((128, PK * RL), fix),
            pl.BlockSpec((128, PK * RL), fix),
            pl.BlockSpec((1, PK * RL), fix),
            pl.BlockSpec((PK * RL, 128), fix),
            pl.BlockSpec((1, 128), fix),
            pl.BlockSpec((128, PK * RL), fix),
            pl.BlockSpec((1, PK * RL), fix),
            pl.BlockSpec((PK * RL, 128), fix),
            pl.BlockSpec((1, 128), fix),
        ],
        out_specs=pl.BlockSpec((NG, T), fix),
        out_shape=jax.ShapeDtypeStruct((NG, T), jnp.float32),
    )(h0f, h1f, p0f, p1f, batchw,
      root_Wbd, conv_bf,
      w1abd, w1bbd, b1f, w1cbd, b1cf,
      w2abd, b2f, w2cbd, b2cf)


def _bd8(m):
    return jnp.kron(jnp.eye(PK, dtype=jnp.float32), m)


def _t8(v):
    return jnp.tile(v, PK).reshape(1, PK * v.shape[0])


@jax.jit
def kernel(x, edge_index, edge_attr, batch,
           enc_W, enc_b,
           edge_W1, edge_b1, edge_W2, edge_b2,
           root_W, conv_b,
           rf1_W1, rf1_b1, rf1_W2, rf1_b2,
           rf2_W1, rf2_b1, rf2_W2, rf2_b2):
    src = edge_index[0]
    dst = edge_index[1]
    zeros_nh = jnp.zeros((N, H), jnp.float32)
    q_sel = jnp.kron(jnp.eye(H, dtype=jnp.float32), jnp.ones((1, H), jnp.float32))
    r_sel = jnp.kron(jnp.ones((H, 1), jnp.float32), jnp.eye(H, dtype=jnp.float32))
    b1r = edge_b1.reshape(1, ML)
    b2r = edge_b2.reshape(1, H * H)

    xf = x.reshape(NB, PK * DF)
    attrf = edge_attr.reshape(EB, 128)

    iw = jnp.arange(N, dtype=jnp.int32)
    nw = PK * (iw % NB) + iw // NB
    batchw = batch[nw].astype(jnp.float32).reshape(N, 1)

    enc_Wbd = _bd8(enc_W)
    enc_bf = _t8(enc_b)
    root_Wbd = _bd8(root_W)
    conv_bf = _t8(conv_b)

    h0f = _enc(xf, enc_Wbd, enc_bf)

    hgf = _sc_gather(h0f.reshape(N, H), src).reshape(EB, 128)
    msgf = _msg(attrf, hgf, edge_W1, b1r, edge_W2, b2r, q_sel, r_sel)
    part = _sc_scatter(msgf.reshape(E, H), dst, zeros_nh).reshape(NC, NB, 128)
    h1f = _upd(part[0], part[1], h0f, root_Wbd, conv_bf)

    hgf2 = _sc_gather(h1f.reshape(N, H), src).reshape(EB, 128)
    msgf2 = _msg(attrf, hgf2, edge_W1, b1r, edge_W2, b2r, q_sel, r_sel)
    part2 = _sc_scatter(msgf2.reshape(E, H), dst, zeros_nh).reshape(NC, NB, 128)

    return _readout(h0f, h1f, part2[0], part2[1], batchw,
                    root_Wbd, conv_bf,
                    _bd8(rf1_W1[:H]), _bd8(rf1_W1[H:]), _t8(rf1_b1),
                    _bd8(rf1_W2), _t8(rf1_b2),
                    _bd8(rf2_W1), _t8(rf2_b1), _bd8(rf2_W2), _t8(rf2_b2))


# speedup vs baseline: 7.0992x; 1.1415x over previous
import functools

import jax
import jax.numpy as jnp
import numpy as np
from jax import lax
from jax.experimental import pallas as pl
from jax.experimental.pallas import tpu as pltpu
from jax.experimental.pallas import tpu_sc as plsc

N = 10000
E = 160000
DF = 128
DE = 16
H = 16
T = 16
NG = 64
ML = 64
RL = 64

NC = 2
NS = 16
NW = NC * NS
EPW = E // NW
RPW = N // NS

PK = 128 // H
TE = 6400
TB = TE // PK
NB = N // PK
EB = E // PK

_mesh = plsc.VectorSubcoreMesh(
    core_axis_name="c", subcore_axis_name="s", num_cores=NC, num_subcores=NS
)


def _unpack(xf, k):
    return jnp.concatenate([xf[:, 16 * b:16 * (b + 1)] for b in range(PK)], axis=0)


def _pack(x, k):
    return jnp.concatenate([x[b * k:(b + 1) * k, :] for b in range(PK)], axis=1)


@functools.partial(
    pl.kernel,
    out_type=jax.ShapeDtypeStruct((E, H), jnp.float32),
    mesh=_mesh,
    compiler_params=pltpu.CompilerParams(use_tc_tiling_on_sc=False),
    scratch_types=[
        pltpu.VMEM((EPW,), jnp.int32),
        pltpu.VMEM((EPW, H), jnp.float32),
        pltpu.SemaphoreType.DMA,
    ],
)
def _sc_gather(h_hbm, src_hbm, out_hbm, idx_v, rows_v, sem):
    wid = lax.axis_index("s") * NC + lax.axis_index("c")
    base = wid * EPW
    pltpu.sync_copy(src_hbm.at[pl.ds(base, EPW)], idx_v)
    pltpu.async_copy(h_hbm.at[idx_v], rows_v, sem).wait()
    pltpu.sync_copy(rows_v, out_hbm.at[pl.ds(base, EPW)])


@functools.partial(
    pl.kernel,
    out_type=jax.ShapeDtypeStruct((NC, N, H), jnp.float32),
    mesh=_mesh,
    compiler_params=pltpu.CompilerParams(use_tc_tiling_on_sc=False),
    scratch_types=[
        pltpu.VMEM((EPW,), jnp.int32),
        pltpu.VMEM((EPW, H), jnp.float32),
        pltpu.VMEM_SHARED((N, H), jnp.float32),
        pltpu.SemaphoreType.DMA,
    ],
)
def _sc_scatter(msg_hbm, dst_hbm, zeros_hbm, out_hbm, idx_v, rows_v, acc_sh, sem):
    c = lax.axis_index("c")
    s = lax.axis_index("s")
    base = (s * NC + c) * EPW
    pltpu.sync_copy(zeros_hbm.at[pl.ds(s * RPW, RPW)], acc_sh.at[pl.ds(s * RPW, RPW)])
    plsc.subcore_barrier()
    pltpu.sync_copy(dst_hbm.at[pl.ds(base, EPW)], idx_v)
    pltpu.sync_copy(msg_hbm.at[pl.ds(base, EPW)], rows_v)
    pltpu.sync_copy(rows_v, acc_sh.at[idx_v], add=True)
    plsc.subcore_barrier()
    pltpu.sync_copy(acc_sh.at[pl.ds(s * RPW, RPW)], out_hbm.at[c, pl.ds(s * RPW, RPW)])


def _enc_body(xf_ref, wbd_ref, bf_ref, o_ref):
    o_ref[...] = xf_ref[...] @ wbd_ref[...] + bf_ref[...]


def _enc(xf, enc_Wbd, enc_bf):
    return pl.pallas_call(
        _enc_body,
        grid=(1,),
        in_specs=[
            pl.BlockSpec((NB, PK * DF), lambda i: (0, 0)),
            pl.BlockSpec((PK * DF, 128), lambda i: (0, 0)),
            pl.BlockSpec((1, 128), lambda i: (0, 0)),
        ],
        out_specs=pl.BlockSpec((NB, 128), lambda i: (0, 0)),
        out_shape=jax.ShapeDtypeStruct((NB, 128), jnp.float32),
    )(xf, enc_Wbd, enc_bf)


def _msg_body(af_ref, hgf_ref, w1_ref, b1_ref, w2_ref, b2_ref, q_ref, r_ref, o_ref):
    attr = _unpack(af_ref[...], TB)
    hg = _unpack(hgf_ref[...], TB)
    z = jnp.maximum(attr @ w1_ref[...] + b1_ref[...], 0.0)
    w = lax.dot_general(z.astype(jnp.bfloat16), w2_ref[...],
                        (((1,), (0,)), ((), ())),
                        preferred_element_type=jnp.float32) + b2_ref[...]
    hrep = hg @ q_ref[...]
    msg = lax.dot_general((hrep * w).astype(jnp.bfloat16), r_ref[...],
                          (((1,), (0,)), ((), ())),
                          preferred_element_type=jnp.float32)
    o_ref[...] = _pack(msg, TB)


def _msg(attrf, hgf, w1, b1, w2, b2, q, r):
    return pl.pallas_call(
        _msg_body,
        grid=(E // TE,),
        in_specs=[
            pl.BlockSpec((TB, 128), lambda i: (i, 0)),
            pl.BlockSpec((TB, 128), lambda i: (i, 0)),
            pl.BlockSpec((DE, ML), lambda i: (0, 0)),
            pl.BlockSpec((1, ML), lambda i: (0, 0)),
            pl.BlockSpec((ML, H * H), lambda i: (0, 0)),
            pl.BlockSpec((1, H * H), lambda i: (0, 0)),
            pl.BlockSpec((H, H * H), lambda i: (0, 0)),
            pl.BlockSpec((H * H, H), lambda i: (0, 0)),
        ],
        out_specs=pl.BlockSpec((TB, 128), lambda i: (i, 0)),
        out_shape=jax.ShapeDtypeStruct((EB, 128), jnp.float32),
    )(attrf, hgf, w1, b1, w2, b2, q, r)


def _upd_body(p0_ref, p1_ref, hf_ref, wbd_ref, bf_ref, o_ref):
    o_ref[...] = (p0_ref[0] + p1_ref[0]
                  + hf_ref[...] @ wbd_ref[...] + bf_ref[...])


def _upd(partf, hf, root_Wbd, conv_bf):
    fix = lambda i: (0, 0)
    return pl.pallas_call(
        _upd_body,
        grid=(1,),
        in_specs=[
            pl.BlockSpec((1, NB, 128), lambda i: (0, 0, 0)),
            pl.BlockSpec((1, NB, 128), lambda i: (1, 0, 0)),
            pl.BlockSpec((NB, 128), fix),
            pl.BlockSpec((128, 128), fix),
            pl.BlockSpec((1, 128), fix),
        ],
        out_specs=pl.BlockSpec((NB, 128), fix),
        out_shape=jax.ShapeDtypeStruct((NB, 128), jnp.float32),
    )(partf, partf, hf, root_Wbd, conv_bf)


def _readout_body(h0_ref, h1_ref, p0_ref, p1_ref, bw_ref,
                  rw_ref, rb_ref,
                  w1a_ref, w1b_ref, b1_ref, w1c_ref, b1c_ref,
                  w2a_ref, b2_ref, w2c_ref, b2c_ref,
                  o_ref):
    h0f = h0_ref[...]
    h2f = p0_ref[0] + p1_ref[0] + h1_ref[...] @ rw_ref[...] + rb_ref[...]
    t1 = jnp.maximum(h0f @ w1a_ref[...] + h2f @ w1b_ref[...] + b1_ref[...], 0.0)
    r1 = t1 @ w1c_ref[...] + b1c_ref[...]
    t2 = jnp.maximum(h2f @ w2a_ref[...] + b2_ref[...], 0.0)
    r2 = t2 @ w2c_ref[...] + b2c_ref[...]
    resf = (1.0 / (1.0 + jnp.exp(-r1))) * r2
    resw = _unpack(resf, NB)
    h0w = _unpack(h0_ref[...], NB)
    maskw = (jnp.sum(h0w, axis=1, keepdims=True) > 0.0).astype(jnp.float32)
    resw = resw * maskw
    gids = lax.broadcasted_iota(jnp.int32, (N, NG), 1).astype(jnp.float32)
    onehot = (bw_ref[...] == gids).astype(jnp.float32)
    o_ref[...] = lax.dot_general(onehot, resw, (((0,), (0,)), ((), ())),
                                 preferred_element_type=jnp.float32)


def _readout(h0f, h1f, partf, batchw,
             root_Wbd, conv_bf,
             w1abd, w1bbd, b1f, w1cbd, b1cf,
             w2abd, b2f, w2cbd, b2cf):
    fix = lambda i: (0, 0)
    return pl.pallas_call(
        _readout_body,
        grid=(1,),
        in_specs=[
            pl.BlockSpec((NB, 128), fix),
            pl.BlockSpec((NB, 128), fix),
            pl.BlockSpec((1, NB, 128), lambda i: (0, 0, 0)),
            pl.BlockSpec((1, NB, 128), lambda i: (1, 0, 0)),
            pl.BlockSpec((N, 1), fix),
            pl.BlockSpec((128, 128), fix),
            pl.BlockSpec((1, 128), fix),
            pl.BlockSpec((128, PK * RL), fix),
            pl.BlockSpec((128, PK * RL), fix),
            pl.BlockSpec((1, PK * RL), fix),
            pl.BlockSpec((PK * RL, 128), fix),
            pl.BlockSpec((1, 128), fix),
            pl.BlockSpec((128, PK * RL), fix),
            pl.BlockSpec((1, PK * RL), fix),
            pl.BlockSpec((PK * RL, 128), fix),
            pl.BlockSpec((1, 128), fix),
        ],
        out_specs=pl.BlockSpec((NG, T), fix),
        out_shape=jax.ShapeDtypeStruct((NG, T), jnp.float32),
    )(h0f, h1f, partf, partf, batchw,
      root_Wbd, conv_bf,
      w1abd, w1bbd, b1f, w1cbd, b1cf,
      w2abd, b2f, w2cbd, b2cf)


def _bd8(m):
    return jnp.kron(jnp.eye(PK, dtype=jnp.float32), m)


def _t8(v):
    return jnp.tile(v, PK).reshape(1, PK * v.shape[0])


@jax.jit
def kernel(x, edge_index, edge_attr, batch,
           enc_W, enc_b,
           edge_W1, edge_b1, edge_W2, edge_b2,
           root_W, conv_b,
           rf1_W1, rf1_b1, rf1_W2, rf1_b2,
           rf2_W1, rf2_b1, rf2_W2, rf2_b2):
    src = edge_index[0]
    dst = edge_index[1]
    zeros_nh = jnp.zeros((N, H), jnp.float32)
    q_sel = jnp.kron(jnp.eye(H, dtype=jnp.float32), jnp.ones((1, H), jnp.float32))
    r_sel = jnp.kron(jnp.ones((H, 1), jnp.float32),
                     jnp.eye(H, dtype=jnp.float32)).astype(jnp.bfloat16)
    w2b = edge_W2.astype(jnp.bfloat16)
    b1r = edge_b1.reshape(1, ML)
    b2r = edge_b2.reshape(1, H * H)

    xf = x.reshape(NB, PK * DF)
    attrf = edge_attr.reshape(EB, 128)

    batchw = batch.reshape(NB, PK).T.astype(jnp.float32).reshape(N, 1)

    enc_Wbd = _bd8(enc_W)
    enc_bf = _t8(enc_b)
    root_Wbd = _bd8(root_W)
    conv_bf = _t8(conv_b)

    h0f = _enc(xf, enc_Wbd, enc_bf)

    hgf = _sc_gather(h0f.reshape(N, H), src).reshape(EB, 128)
    msgf = _msg(attrf, hgf, edge_W1, b1r, w2b, b2r, q_sel, r_sel)
    part = _sc_scatter(msgf.reshape(E, H), dst, zeros_nh).reshape(NC, NB, 128)
    h1f = _upd(part, h0f, root_Wbd, conv_bf)

    hgf2 = _sc_gather(h1f.reshape(N, H), src).reshape(EB, 128)
    msgf2 = _msg(attrf, hgf2, edge_W1, b1r, w2b, b2r, q_sel, r_sel)
    part2 = _sc_scatter(msgf2.reshape(E, H), dst, zeros_nh).reshape(NC, NB, 128)

    return _readout(h0f, h1f, part2, batchw,
                    root_Wbd, conv_bf,
                    _bd8(rf1_W1[:H]), _bd8(rf1_W1[H:]), _t8(rf1_b1),
                    _bd8(rf1_W2), _t8(rf1_b2),
                    _bd8(rf2_W1), _t8(rf2_b1), _bd8(rf2_W2), _t8(rf2_b2))
